# baseline (speedup 1.0000x reference)
import jax
import jax.numpy as jnp
from jax import lax
from jax.experimental import pallas as pl
from jax.experimental.pallas import tpu as pltpu
from jax.experimental.pallas import tpu_sc as plsc

HIDDEN = 2048
NUM_EXPERTS = 64
TOPK = 8
CPU_K = 24
LANES = 16
NUM_WORKERS = 32


def _matmul_body(x_ref, w_ref, o_ref):
    o_ref[...] = lax.dot_general(
        x_ref[...], w_ref[...], (((1,), (1,)), ((), ())),
        precision=lax.Precision.DEFAULT,
        preferred_element_type=jnp.float32)


def _router_logits(hs2, wg):
    m = hs2.shape[0]
    bm = 1024
    return pl.pallas_call(
        _matmul_body,
        grid=(m // bm,),
        in_specs=[pl.BlockSpec((bm, HIDDEN), lambda i: (i, 0)),
                  pl.BlockSpec((NUM_EXPERTS, HIDDEN), lambda i: (0, 0))],
        out_specs=pl.BlockSpec((bm, NUM_EXPERTS), lambda i: (i, 0)),
        out_shape=jax.ShapeDtypeStruct((m, NUM_EXPERTS), jnp.float32),
    )(hs2, wg)


def _beats(ka, ia, kb, ib):
    return (ka > kb) | ((ka == kb) & (ia < ib))


def _sc_body(rows_per_worker, chunk_rows, logits_hbm, mix_hbm, lv, mv):
    wid = lax.axis_index("s") * 2 + lax.axis_index("c")
    base = wid * rows_per_worker

    iota = lax.iota(jnp.int32, LANES)
    mask8 = iota < TOPK

    def row(r, carry):
        ks, js = [], []
        for c in range(4):
            v = lv[r, pl.ds(c * LANES, LANES)]
            k_, j_ = plsc.sort_key_val(v, iota + (LANES * c), descending=True)
            ks.append(k_)
            js.append(j_)

        def merge2(k0, i0, k1, i1):
            brk = lax.rev(k1, (0,))
            bri = lax.rev(i1, (0,))
            win = _beats(k0, i0, brk, bri)
            wk = jnp.where(win, k0, brk)
            wi = jnp.where(win, i0, bri)
            lk = jnp.where(win, brk, k0)
            li = jnp.where(win, bri, i0)
            wk, wi = plsc.sort_key_val(wk, wi, descending=True)
            lk, li = plsc.sort_key_val(lk, li, descending=True)
            return wk, wi, lk, li

        a0k, a0i, a1k, a1i = merge2(ks[0], js[0], ks[1], js[1])
        b0k, b0i, b1k, b1i = merge2(ks[2], js[2], ks[3], js[3])

        br0k = lax.rev(b1k, (0,))
        br0i = lax.rev(b1i, (0,))
        br1k = lax.rev(b0k, (0,))
        br1i = lax.rev(b0i, (0,))
        w0 = _beats(a0k, a0i, br0k, br0i)
        h0k = jnp.where(w0, a0k, br0k)
        h0i = jnp.where(w0, a0i, br0i)
        w1 = _beats(a1k, a1i, br1k, br1i)
        h1k = jnp.where(w1, a1k, br1k)
        h1i = jnp.where(w1, a1i, br1i)
        wd = _beats(h0k, h0i, h1k, h1i)
        t0k = jnp.where(wd, h0k, h1k)
        t0i = jnp.where(wd, h0i, h1i)
        t1i = jnp.where(wd, h1i, h0i)
        t1k = jnp.where(wd, h1k, h0k)
        t0k, t0i = plsc.sort_key_val(t0k, t0i, descending=True)
        t1k, t1i = plsc.sort_key_val(t1k, t1i, descending=True)

        m = jnp.max(t0k)
        e = jnp.exp(t0k - m)
        e8 = jnp.where(mask8, e, 0.0)
        s = jnp.sum(e8)
        mv[r, pl.ds(0, LANES)] = t0i
        mv[r, pl.ds(LANES, LANES)] = t1i
        mv[r, pl.ds(2 * LANES, LANES)] = plsc.bitcast(e8 / s, jnp.int32)
        return carry

    def row_pair(i, carry):
        row(2 * i, carry)
        row(2 * i + 1, carry)
        return carry

    for chunk in range(rows_per_worker // chunk_rows):
        cbase = base + chunk * chunk_rows
        pltpu.sync_copy(logits_hbm.at[pl.ds(cbase, chunk_rows)], lv)
        lax.fori_loop(0, chunk_rows // 2, row_pair, 0)
        pltpu.sync_copy(mv, mix_hbm.at[pl.ds(cbase, chunk_rows)])


def _sc_topk(logits):
    rows = logits.shape[0]
    rpw = rows // NUM_WORKERS
    cr = 256
    mesh = plsc.VectorSubcoreMesh(core_axis_name="c", subcore_axis_name="s")
    f = pl.kernel(
        lambda *args: _sc_body(rpw, cr, *args),
        out_type=jax.ShapeDtypeStruct((rows, 3 * LANES), jnp.int32),
        mesh=mesh,
        scratch_types=[pltpu.VMEM((cr, NUM_EXPERTS), jnp.float32),
                       pltpu.VMEM((cr, 3 * LANES), jnp.int32)],
        compiler_params=pltpu.CompilerParams(needs_layout_passes=False),
    )
    return f(logits)


def _repack_body(x_ref, ei_ref, cpu_ref, ew_ref):
    x = x_ref[...]
    bm = x.shape[0]
    ei_ref[...] = lax.slice(x, (0, 0), (bm, TOPK))
    cpu_ref[...] = lax.slice(x, (0, TOPK), (bm, TOPK + CPU_K))
    ew_ref[...] = lax.bitcast_convert_type(
        lax.slice(x, (0, 2 * LANES), (bm, 2 * LANES + TOPK)), jnp.float32)


def _repack(mix):
    rows = mix.shape[0]
    bm = 4096
    return pl.pallas_call(
        _repack_body,
        grid=(rows // bm,),
        in_specs=[pl.BlockSpec((bm, 3 * LANES), lambda i: (i, 0))],
        out_specs=[pl.BlockSpec((bm, TOPK), lambda i: (i, 0)),
                   pl.BlockSpec((bm, CPU_K), lambda i: (i, 0)),
                   pl.BlockSpec((bm, TOPK), lambda i: (i, 0))],
        out_shape=(jax.ShapeDtypeStruct((rows, TOPK), jnp.int32),
                   jax.ShapeDtypeStruct((rows, CPU_K), jnp.int32),
                   jax.ShapeDtypeStruct((rows, TOPK), jnp.float32)),
    )(mix)


def kernel(hidden_states, wg_weight):
    b, s, h = hidden_states.shape
    hs2 = hidden_states.reshape(b * s, h)
    logits = _router_logits(hs2, wg_weight)
    mix = _sc_topk(logits)
    expert_index, cpu_expert_index, expert_weight = _repack(mix)
    return expert_weight, expert_index, cpu_expert_index, expert_index


# speedup vs baseline: 1.0206x; 1.0206x over previous
import jax
import jax.numpy as jnp
from jax import lax
from jax.experimental import pallas as pl
from jax.experimental.pallas import tpu as pltpu
from jax.experimental.pallas import tpu_sc as plsc

HIDDEN = 2048
NUM_EXPERTS = 64
TOPK = 8
CPU_K = 24
LANES = 16
NUM_WORKERS = 32


def _matmul_body(x_ref, w_ref, o_ref):
    o_ref[...] = lax.dot_general(
        x_ref[...], w_ref[...], (((1,), (1,)), ((), ())),
        precision=lax.Precision.DEFAULT,
        preferred_element_type=jnp.float32)


def _router_logits(hs2, wg):
    m = hs2.shape[0]
    bm = 2048
    return pl.pallas_call(
        _matmul_body,
        grid=(m // bm,),
        in_specs=[pl.BlockSpec((bm, HIDDEN), lambda i: (i, 0)),
                  pl.BlockSpec((NUM_EXPERTS, HIDDEN), lambda i: (0, 0))],
        out_specs=pl.BlockSpec((bm, NUM_EXPERTS), lambda i: (i, 0)),
        out_shape=jax.ShapeDtypeStruct((m, NUM_EXPERTS), jnp.float32),
    )(hs2, wg)


def _beats(ka, ia, kb, ib):
    return (ka > kb) | ((ka == kb) & (ia < ib))


def _sc_body(rows_per_worker, chunk_rows, logits_hbm, w_hbm, idx_hbm, lv, wv, iv):
    wid = lax.axis_index("s") * 2 + lax.axis_index("c")
    base = wid * rows_per_worker

    iota = lax.iota(jnp.int32, LANES)
    mask8 = iota < TOPK

    def row(r, carry):
        ks, js = [], []
        for c in range(4):
            v = lv[r, pl.ds(c * LANES, LANES)]
            k_, j_ = plsc.sort_key_val(v, iota + (LANES * c), descending=True)
            ks.append(k_)
            js.append(j_)

        def merge2(k0, i0, k1, i1):
            brk = lax.rev(k1, (0,))
            bri = lax.rev(i1, (0,))
            win = _beats(k0, i0, brk, bri)
            wk = jnp.where(win, k0, brk)
            wi = jnp.where(win, i0, bri)
            lk = jnp.where(win, brk, k0)
            li = jnp.where(win, bri, i0)
            wk, wi = plsc.sort_key_val(wk, wi, descending=True)
            lk, li = plsc.sort_key_val(lk, li, descending=True)
            return wk, wi, lk, li

        a0k, a0i, a1k, a1i = merge2(ks[0], js[0], ks[1], js[1])
        b0k, b0i, b1k, b1i = merge2(ks[2], js[2], ks[3], js[3])

        br0k = lax.rev(b1k, (0,))
        br0i = lax.rev(b1i, (0,))
        br1k = lax.rev(b0k, (0,))
        br1i = lax.rev(b0i, (0,))
        w0 = _beats(a0k, a0i, br0k, br0i)
        h0k = jnp.where(w0, a0k, br0k)
        h0i = jnp.where(w0, a0i, br0i)
        w1 = _beats(a1k, a1i, br1k, br1i)
        h1k = jnp.where(w1, a1k, br1k)
        h1i = jnp.where(w1, a1i, br1i)
        wd = _beats(h0k, h0i, h1k, h1i)
        t0k = jnp.where(wd, h0k, h1k)
        t0i = jnp.where(wd, h0i, h1i)
        t1i = jnp.where(wd, h1i, h0i)
        t1k = jnp.where(wd, h1k, h0k)
        t0k, t0i = plsc.sort_key_val(t0k, t0i, descending=True)
        t1k, t1i = plsc.sort_key_val(t1k, t1i, descending=True)

        m = jnp.max(t0k)
        e = jnp.exp(t0k - m)
        e8 = jnp.where(mask8, e, 0.0)
        s = jnp.sum(e8)
        wv[r, pl.ds(0, LANES)] = e8 / s
        iv[r, pl.ds(0, LANES)] = t0i
        iv[r, pl.ds(LANES, LANES)] = t1i
        return carry

    def row_pair(i, carry):
        row(2 * i, carry)
        row(2 * i + 1, carry)
        return carry

    for chunk in range(rows_per_worker // chunk_rows):
        cbase = base + chunk * chunk_rows
        pltpu.sync_copy(logits_hbm.at[pl.ds(cbase, chunk_rows)], lv)
        lax.fori_loop(0, chunk_rows // 2, row_pair, 0)
        pltpu.sync_copy(wv, w_hbm.at[pl.ds(cbase, chunk_rows)])
        pltpu.sync_copy(iv, idx_hbm.at[pl.ds(cbase, chunk_rows)])


def _sc_topk(logits):
    rows = logits.shape[0]
    rpw = rows // NUM_WORKERS
    cr = 256
    mesh = plsc.VectorSubcoreMesh(core_axis_name="c", subcore_axis_name="s")
    f = pl.kernel(
        lambda *args: _sc_body(rpw, cr, *args),
        out_type=(jax.ShapeDtypeStruct((rows, LANES), jnp.float32),
                  jax.ShapeDtypeStruct((rows, 2 * LANES), jnp.int32)),
        mesh=mesh,
        scratch_types=[pltpu.VMEM((cr, NUM_EXPERTS), jnp.float32),
                       pltpu.VMEM((cr, LANES), jnp.float32),
                       pltpu.VMEM((cr, 2 * LANES), jnp.int32)],
        compiler_params=pltpu.CompilerParams(needs_layout_passes=False),
    )
    return f(logits)


def kernel(hidden_states, wg_weight):
    b, s, h = hidden_states.shape
    hs2 = hidden_states.reshape(b * s, h)
    logits = _router_logits(hs2, wg_weight)
    w_p, idx_p = _sc_topk(logits)
    expert_weight = w_p[:, :TOPK]
    expert_index = idx_p[:, :TOPK]
    cpu_expert_index = idx_p[:, TOPK:TOPK + CPU_K]
    return expert_weight, expert_index, cpu_expert_index, expert_index


# speedup vs baseline: 1.1779x; 1.1541x over previous
import jax
import jax.numpy as jnp
from jax import lax
from jax.experimental import pallas as pl
from jax.experimental.pallas import tpu as pltpu
from jax.experimental.pallas import tpu_sc as plsc

HIDDEN = 2048
NUM_EXPERTS = 64
TOPK = 8
CPU_K = 24
LANES = 16
NUM_WORKERS = 32


def _matmul_body(x_ref, w_ref, o_ref):
    o_ref[...] = lax.dot_general(
        x_ref[...], w_ref[...], (((1,), (1,)), ((), ())),
        precision=lax.Precision.DEFAULT,
        preferred_element_type=jnp.float32)


def _router_logits(hs2, wg, chunk=0, nchunks=1):
    m = hs2.shape[0] // nchunks
    bm = 1024
    nb = m // bm
    return pl.pallas_call(
        _matmul_body,
        grid=(nb,),
        in_specs=[pl.BlockSpec((bm, HIDDEN), lambda i, c=chunk, n=nb: (i + c * n, 0)),
                  pl.BlockSpec((NUM_EXPERTS, HIDDEN), lambda i: (0, 0))],
        out_specs=pl.BlockSpec((bm, NUM_EXPERTS), lambda i: (i, 0)),
        out_shape=jax.ShapeDtypeStruct((m, NUM_EXPERTS), jnp.float32),
    )(hs2, wg)


def _beats(ka, ia, kb, ib):
    return (ka > kb) | ((ka == kb) & (ia < ib))


def _sc_body(rows_per_worker, chunk_rows, logits_hbm, w_hbm, idx_hbm, lv, wv, iv):
    wid = lax.axis_index("s") * 2 + lax.axis_index("c")
    base = wid * rows_per_worker

    iota = lax.iota(jnp.int32, LANES)
    mask8 = iota < TOPK

    def row(r, carry):
        ks, js = [], []
        for c in range(4):
            v = lv[r, pl.ds(c * LANES, LANES)]
            k_, j_ = plsc.sort_key_val(v, iota + (LANES * c), descending=True)
            ks.append(k_)
            js.append(j_)

        def merge2(k0, i0, k1, i1):
            brk = lax.rev(k1, (0,))
            bri = lax.rev(i1, (0,))
            win = _beats(k0, i0, brk, bri)
            wk = jnp.where(win, k0, brk)
            wi = jnp.where(win, i0, bri)
            lk = jnp.where(win, brk, k0)
            li = jnp.where(win, bri, i0)
            wk, wi = plsc.sort_key_val(wk, wi, descending=True)
            lk, li = plsc.sort_key_val(lk, li, descending=True)
            return wk, wi, lk, li

        a0k, a0i, a1k, a1i = merge2(ks[0], js[0], ks[1], js[1])
        b0k, b0i, b1k, b1i = merge2(ks[2], js[2], ks[3], js[3])

        br0k = lax.rev(b1k, (0,))
        br0i = lax.rev(b1i, (0,))
        br1k = lax.rev(b0k, (0,))
        br1i = lax.rev(b0i, (0,))
        w0 = _beats(a0k, a0i, br0k, br0i)
        h0k = jnp.where(w0, a0k, br0k)
        h0i = jnp.where(w0, a0i, br0i)
        w1 = _beats(a1k, a1i, br1k, br1i)
        h1k = jnp.where(w1, a1k, br1k)
        h1i = jnp.where(w1, a1i, br1i)
        wd = _beats(h0k, h0i, h1k, h1i)
        t0k = jnp.where(wd, h0k, h1k)
        t0i = jnp.where(wd, h0i, h1i)
        t1i = jnp.where(wd, h1i, h0i)
        t1k = jnp.where(wd, h1k, h0k)
        t0k, t0i = plsc.sort_key_val(t0k, t0i, descending=True)
        t1k, t1i = plsc.sort_key_val(t1k, t1i, descending=True)

        m = jnp.max(t0k)
        e = jnp.exp(t0k - m)
        e8 = jnp.where(mask8, e, 0.0)
        s = jnp.sum(e8)
        wv[r, pl.ds(0, LANES)] = e8 / s
        iv[r, pl.ds(0, LANES)] = t0i
        iv[r, pl.ds(LANES, LANES)] = t1i
        return carry

    def row_pair(i, carry):
        row(2 * i, carry)
        row(2 * i + 1, carry)
        return carry

    for chunk in range(rows_per_worker // chunk_rows):
        cbase = base + chunk * chunk_rows
        pltpu.sync_copy(logits_hbm.at[pl.ds(cbase, chunk_rows)], lv)
        lax.fori_loop(0, chunk_rows // 2, row_pair, 0)
        pltpu.sync_copy(wv, w_hbm.at[pl.ds(cbase, chunk_rows)])
        pltpu.sync_copy(iv, idx_hbm.at[pl.ds(cbase, chunk_rows)])


def _sc_topk(logits):
    rows = logits.shape[0]
    rpw = rows // NUM_WORKERS
    cr = 256
    mesh = plsc.VectorSubcoreMesh(core_axis_name="c", subcore_axis_name="s")
    f = pl.kernel(
        lambda *args: _sc_body(rpw, cr, *args),
        out_type=(jax.ShapeDtypeStruct((rows, LANES), jnp.float32),
                  jax.ShapeDtypeStruct((rows, 2 * LANES), jnp.int32)),
        mesh=mesh,
        scratch_types=[pltpu.VMEM((cr, NUM_EXPERTS), jnp.float32),
                       pltpu.VMEM((cr, LANES), jnp.float32),
                       pltpu.VMEM((cr, 2 * LANES), jnp.int32)],
        compiler_params=pltpu.CompilerParams(needs_layout_passes=False),
    )
    return f(logits)


def kernel(hidden_states, wg_weight):
    b, s, h = hidden_states.shape
    hs2 = hidden_states.reshape(b * s, h)
    nchunks = 2
    w_parts, idx_parts = [], []
    for c in range(nchunks):
        logits_c = _router_logits(hs2, wg_weight, chunk=c, nchunks=nchunks)
        w_c, idx_c = _sc_topk(logits_c)
        w_parts.append(w_c)
        idx_parts.append(idx_c)
    w_p = jnp.concatenate(w_parts, axis=0)
    idx_p = jnp.concatenate(idx_parts, axis=0)
    expert_weight = w_p[:, :TOPK]
    expert_index = idx_p[:, :TOPK]
    cpu_expert_index = idx_p[:, TOPK:TOPK + CPU_K]
    return expert_weight, expert_index, cpu_expert_index, expert_index


# speedup vs baseline: 1.2345x; 1.0481x over previous
import jax
import jax.numpy as jnp
from jax import lax
from jax.experimental import pallas as pl
from jax.experimental.pallas import tpu as pltpu
from jax.experimental.pallas import tpu_sc as plsc

HIDDEN = 2048
NUM_EXPERTS = 64
TOPK = 8
CPU_K = 24
LANES = 16
NUM_WORKERS = 32


def _matmul_body(x_ref, w_ref, o_ref):
    o_ref[...] = lax.dot_general(
        x_ref[...], w_ref[...], (((1,), (1,)), ((), ())),
        precision=lax.Precision.DEFAULT,
        preferred_element_type=jnp.float32)


def _router_logits(hs2, wg, chunk=0, nchunks=1):
    m = hs2.shape[0] // nchunks
    bm = 1024
    nb = m // bm
    return pl.pallas_call(
        _matmul_body,
        grid=(nb,),
        in_specs=[pl.BlockSpec((bm, HIDDEN), lambda i, c=chunk, n=nb: (i + c * n, 0)),
                  pl.BlockSpec((NUM_EXPERTS, HIDDEN), lambda i: (0, 0))],
        out_specs=pl.BlockSpec((bm, NUM_EXPERTS), lambda i: (i, 0)),
        out_shape=jax.ShapeDtypeStruct((m, NUM_EXPERTS), jnp.float32),
    )(hs2, wg)


def _beats(ka, ia, kb, ib):
    return (ka > kb) | ((ka == kb) & (ia < ib))


def _sc_body(rows_per_worker, chunk_rows, logits_hbm, w_hbm, idx_hbm, lv, wv, iv):
    wid = lax.axis_index("s") * 2 + lax.axis_index("c")
    base = wid * rows_per_worker

    iota = lax.iota(jnp.int32, LANES)
    mask8 = iota < TOPK

    def row(r, carry):
        ks, js = [], []
        for c in range(4):
            v = lv[r, pl.ds(c * LANES, LANES)]
            k_, j_ = plsc.sort_key_val(v, iota + (LANES * c), descending=True)
            ks.append(k_)
            js.append(j_)

        def merge2(k0, i0, k1, i1):
            brk = lax.rev(k1, (0,))
            bri = lax.rev(i1, (0,))
            win = _beats(k0, i0, brk, bri)
            wk = jnp.where(win, k0, brk)
            wi = jnp.where(win, i0, bri)
            lk = jnp.where(win, brk, k0)
            li = jnp.where(win, bri, i0)
            wk, wi = plsc.sort_key_val(wk, wi, descending=True)
            lk, li = plsc.sort_key_val(lk, li, descending=True)
            return wk, wi, lk, li

        a0k, a0i, a1k, a1i = merge2(ks[0], js[0], ks[1], js[1])
        b0k, b0i, b1k, b1i = merge2(ks[2], js[2], ks[3], js[3])

        br0k = lax.rev(b1k, (0,))
        br0i = lax.rev(b1i, (0,))
        br1k = lax.rev(b0k, (0,))
        br1i = lax.rev(b0i, (0,))
        w0 = _beats(a0k, a0i, br0k, br0i)
        h0k = jnp.where(w0, a0k, br0k)
        h0i = jnp.where(w0, a0i, br0i)
        w1 = _beats(a1k, a1i, br1k, br1i)
        h1k = jnp.where(w1, a1k, br1k)
        h1i = jnp.where(w1, a1i, br1i)
        wd = _beats(h0k, h0i, h1k, h1i)
        t0k = jnp.where(wd, h0k, h1k)
        t0i = jnp.where(wd, h0i, h1i)
        t1i = jnp.where(wd, h1i, h0i)
        t1k = jnp.where(wd, h1k, h0k)
        t0k, t0i = plsc.sort_key_val(t0k, t0i, descending=True)
        t1k, t1i = plsc.sort_key_val(t1k, t1i, descending=True)

        m = jnp.max(t0k)
        e = jnp.exp(t0k - m)
        e8 = jnp.where(mask8, e, 0.0)
        s = jnp.sum(e8)
        wv[r, pl.ds(0, LANES)] = e8 / s
        iv[r, pl.ds(0, LANES)] = t0i
        iv[r, pl.ds(LANES, LANES)] = t1i
        return carry

    def row_pair(i, carry):
        row(2 * i, carry)
        row(2 * i + 1, carry)
        return carry

    for chunk in range(rows_per_worker // chunk_rows):
        cbase = base + chunk * chunk_rows
        pltpu.sync_copy(logits_hbm.at[pl.ds(cbase, chunk_rows)], lv)
        lax.fori_loop(0, chunk_rows // 2, row_pair, 0)
        pltpu.sync_copy(wv, w_hbm.at[pl.ds(cbase, chunk_rows)])
        pltpu.sync_copy(iv, idx_hbm.at[pl.ds(cbase, chunk_rows)])


def _sc_topk(logits):
    rows = logits.shape[0]
    rpw = rows // NUM_WORKERS
    cr = 256
    mesh = plsc.VectorSubcoreMesh(core_axis_name="c", subcore_axis_name="s")
    f = pl.kernel(
        lambda *args: _sc_body(rpw, cr, *args),
        out_type=(jax.ShapeDtypeStruct((rows, LANES), jnp.float32),
                  jax.ShapeDtypeStruct((rows, 2 * LANES), jnp.int32)),
        mesh=mesh,
        scratch_types=[pltpu.VMEM((cr, NUM_EXPERTS), jnp.float32),
                       pltpu.VMEM((cr, LANES), jnp.float32),
                       pltpu.VMEM((cr, 2 * LANES), jnp.int32)],
        compiler_params=pltpu.CompilerParams(needs_layout_passes=False),
    )
    return f(logits)


def kernel(hidden_states, wg_weight):
    b, s, h = hidden_states.shape
    hs2 = hidden_states.reshape(b * s, h)
    nchunks = 4
    w_parts, idx_parts = [], []
    for c in range(nchunks):
        logits_c = _router_logits(hs2, wg_weight, chunk=c, nchunks=nchunks)
        w_c, idx_c = _sc_topk(logits_c)
        w_parts.append(w_c)
        idx_parts.append(idx_c)
    w_p = jnp.concatenate(w_parts, axis=0)
    idx_p = jnp.concatenate(idx_parts, axis=0)
    expert_weight = w_p[:, :TOPK]
    expert_index = idx_p[:, :TOPK]
    cpu_expert_index = idx_p[:, TOPK:TOPK + CPU_K]
    return expert_weight, expert_index, cpu_expert_index, expert_index
